# baseline (speedup 1.0000x reference)
import functools

import jax
import jax.numpy as jnp
from jax import lax
from jax.experimental import pallas as pl
from jax.experimental.pallas import tpu as pltpu
from jax.experimental.pallas import tpu_sc as plsc

N = 10000
K = 16
B = 4
EMB = 128
FD = 64
C = 8.0
MU = 0.5
CG_ITERS = 10

NPAD = 10240
ROW = B * FD
T2 = 32
NT2 = NPAD // T2
T3 = 16
NT3 = NPAD // T3
CH = 16
NCHUNK = NT2 // CH
K1_BLK = 256


def _k1_body(emb_ref, x_ref, w1t_ref, w0_ref, b_ref, out_ref):
    g = jnp.dot(emb_ref[...], w1t_ref[...],
                preferred_element_type=jnp.float32) + b_ref[...]
    cols = []
    for b in range(B):
        fb = g + x_ref[b][:, None] * w0_ref[...]
        cols.append(jnp.where(fb >= 0, fb, 0.2 * fb))
    out_ref[...] = jnp.concatenate(cols, axis=1).astype(jnp.bfloat16)


def _k1(emb_pad, x_pad8, w1t, w0, bias):
    return pl.pallas_call(
        _k1_body,
        grid=(NPAD // K1_BLK,),
        in_specs=[
            pl.BlockSpec((K1_BLK, EMB), lambda i: (i, 0)),
            pl.BlockSpec((8, K1_BLK), lambda i: (0, i)),
            pl.BlockSpec((EMB, FD), lambda i: (0, 0)),
            pl.BlockSpec((1, FD), lambda i: (0, 0)),
            pl.BlockSpec((1, FD), lambda i: (0, 0)),
        ],
        out_specs=pl.BlockSpec((K1_BLK, ROW), lambda i: (i, 0)),
        out_shape=jax.ShapeDtypeStruct((NPAD, ROW), jnp.bfloat16),
    )(emb_pad, x_pad8, w1t, w0, bias)


_MESH2 = plsc.VectorSubcoreMesh(core_axis_name="c", subcore_axis_name="s")


@functools.partial(
    pl.kernel,
    out_type=jax.ShapeDtypeStruct((T2, K, NT2), jnp.float32),
    mesh=_MESH2,
    compiler_params=pltpu.CompilerParams(use_tc_tiling_on_sc=False, needs_layout_passes=False),
    scratch_types=[
        pltpu.VMEM((NCHUNK, CH * K), jnp.int32),
        pltpu.VMEM((3, CH * K, ROW), jnp.bfloat16),
        pltpu.VMEM((3, CH, ROW), jnp.bfloat16),
        pltpu.VMEM((K, NT2), jnp.float32),
        pltpu.VMEM((16,), jnp.float32),
        pltpu.VMEM((B * K,), jnp.float32),
        pltpu.SemaphoreType.DMA((3, 2)),
        pltpu.SemaphoreType.DMA((3,)),
    ],
)
def _k2(ftab, nlg, params, w2, nlbuf, nbbuf, cbuf, wtbuf, pvec, d2buf,
        semn, semc):
    wid = lax.axis_index("s") * 2 + lax.axis_index("c")
    n0 = wid * NT2
    pltpu.sync_copy(nlg.at[wid], nlbuf)
    pltpu.sync_copy(params, pvec)
    inv2t = pvec[...][0]
    lane = lax.iota(jnp.int32, 16)
    m15 = lane == 15
    H = CH * K // 2

    def start(chunk, sl):
        for h in range(2):
            pltpu.make_async_copy(
                ftab.at[nlbuf.at[chunk, pl.ds(h * H, H)]],
                nbbuf.at[sl, pl.ds(h * H, H)], semn.at[sl, h]).start()
        pltpu.make_async_copy(
            ftab.at[pl.ds(n0 + chunk * CH, CH)], cbuf.at[sl],
            semc.at[sl]).start()

    def wait(chunk, sl):
        for h in range(2):
            pltpu.make_async_copy(
                ftab.at[nlbuf.at[chunk, pl.ds(h * H, H)]],
                nbbuf.at[sl, pl.ds(h * H, H)], semn.at[sl, h]).wait()
        pltpu.make_async_copy(
            ftab.at[pl.ds(n0 + chunk * CH, CH)], cbuf.at[sl],
            semc.at[sl]).wait()

    start(0, 0)
    start(1, 1)
    start(2, 2)

    def chunk_body(chunk, _):
        sl = lax.rem(chunk, 3)
        wait(chunk, sl)

        def node_body(i, _):
            cv = []
            for g in range(ROW // 32):
                cv.extend(plsc.unpack(
                    cbuf[sl, i, pl.ds(g * 32, 32)],
                    format=plsc.PackFormat.INTERLEAVED))
            row0 = i * K
            for k in range(K):
                for b in range(B):
                    acc = None
                    for g in range(2):
                        na, nb_ = plsc.unpack(
                            nbbuf[sl, row0 + k, pl.ds(b * 64 + g * 32, 32)],
                            format=plsc.PackFormat.INTERLEAVED)
                        for h, nv in enumerate((na, nb_)):
                            dv = nv - cv[b * 4 + g * 2 + h]
                            acc = dv * dv if acc is None else acc + dv * dv
                    cum = plsc.cumsum(acc)
                    plsc.store_scatter(
                        d2buf, [jnp.full((16,), b * K + k, jnp.int32)],
                        cum, mask=m15)
            es = None
            for b in range(B):
                e = jnp.exp(d2buf[pl.ds(b * K, 16)] * inv2t)
                es = e if es is None else es + e
            w_row = es * 0.25
            nloc = chunk * CH + i
            w_row = jnp.where(n0 + nloc < N, w_row, 0.0)
            plsc.store_scatter(
                wtbuf, [lane, jnp.full((16,), nloc, jnp.int32)], w_row)
            return 0

        lax.fori_loop(0, CH, node_body, 0)

        @pl.when(chunk + 3 < NCHUNK)
        def _():
            start(chunk + 3, sl)
        return 0

    lax.fori_loop(0, NCHUNK, chunk_body, 0)
    pltpu.sync_copy(wtbuf, w2.at[wid])


_MESH3 = plsc.VectorSubcoreMesh(core_axis_name="c", subcore_axis_name="s")


@functools.partial(
    pl.kernel,
    out_type=jax.ShapeDtypeStruct((B, NPAD), jnp.float32),
    mesh=_MESH3,
    compiler_params=pltpu.CompilerParams(use_tc_tiling_on_sc=False, needs_layout_passes=False),
    scratch_types=[
        pltpu.VMEM((K, NT3), jnp.float32),
        pltpu.VMEM((K, NT3), jnp.int32),
        pltpu.VMEM((NPAD * B,), jnp.float32),
        pltpu.VMEM((B, NT3), jnp.float32),
        pltpu.VMEM((B, NT3), jnp.float32),
        pltpu.VMEM((B, NT3), jnp.float32),
        pltpu.VMEM((B, NT3), jnp.float32),
        pltpu.VMEM((NT3,), jnp.float32),
        pltpu.VMEM((NT3 * B,), jnp.float32),
        pltpu.VMEM((16,), jnp.float32),
        pltpu.VMEM((T3, 16), jnp.float32),
        pltpu.VMEM_SHARED((2, NPAD * B), jnp.float32),
        pltpu.VMEM_SHARED((T3, 16), jnp.float32),
        pltpu.VMEM_SHARED((T3, 16), jnp.float32),
    ],
)
def _k3(x_hbm, xT_hbm, w2_hbm, nl4_hbm, xout,
        wbuf, nlbuf, pfull, ps, rs_, xs, ap, degbuf, ptbuf, rowbuf, redbuf,
        p_sp, partA, partB):
    cid = lax.axis_index("c")
    t = lax.axis_index("s")
    n0 = t * NT3
    lane = lax.iota(jnp.int32, 16)
    zero16 = jnp.zeros((16,), jnp.float32)
    NG = NT3 // 16

    pltpu.sync_copy(w2_hbm.at[2 * t], wbuf.at[:, pl.ds(0, NT2)])
    pltpu.sync_copy(w2_hbm.at[2 * t + 1], wbuf.at[:, pl.ds(NT2, NT2)])
    pltpu.sync_copy(nl4_hbm.at[t], nlbuf)
    pltpu.sync_copy(xT_hbm, pfull)
    pltpu.sync_copy(x_hbm.at[:, pl.ds(n0, NT3)], ps)
    pltpu.sync_copy(x_hbm.at[:, pl.ds(n0, NT3)], rs_)

    def zero_body(g, _):
        for b in range(B):
            xs[b, pl.ds(g * 16, 16)] = zero16
        return 0
    lax.fori_loop(0, NG, zero_body, 0)

    def deg_body(g, carry):
        sw2, sd2 = carry
        dv = zero16
        for k in range(K):
            wv = wbuf[k, pl.ds(g * 16, 16)]
            dv = dv + wv
            sw2 = sw2 + wv * wv
        degbuf[pl.ds(g * 16, 16)] = dv
        return (sw2, sd2 + dv * dv)
    sw2, sd2 = lax.fori_loop(0, NG, deg_body, (zero16, zero16))

    def global_reduce(vals, part_sp):
        v = zero16
        for b, val in enumerate(vals):
            v = v + jnp.where(lane == b, val, 0.0)
        rowbuf[...] = v
        pltpu.sync_copy(rowbuf, part_sp.at[t])
        plsc.subcore_barrier()
        pltpu.sync_copy(part_sp, redbuf)
        return [
            jnp.full((16,), jnp.sum(plsc.load_gather(
                redbuf, [lane, jnp.full((16,), b, jnp.int32)])))
            for b in range(len(vals))
        ]

    l2a, l2b = global_reduce([jnp.sum(sw2), jnp.sum(sd2)], partA)
    y = l2a + l2b
    ib = plsc.bitcast(y, jnp.int32)
    ib = 0x5F3759DF - lax.shift_right_logical(ib, 1)
    rt = plsc.bitcast(ib, jnp.float32)
    for _ in range(4):
        rt = rt * (1.5 - 0.5 * y * rt * rt)
    mus = (MU * C) * rt

    def dot_slices(b1, b2):
        def rb(g, acs):
            out = []
            for b in range(B):
                out.append(acs[b] + b1[b, pl.ds(g * 16, 16)] *
                           b2[b, pl.ds(g * 16, 16)])
            return tuple(out)
        acs = lax.fori_loop(0, NG, rb, (zero16,) * B)
        return [jnp.sum(a) for a in acs]

    rs_vecs = global_reduce(dot_slices(rs_, rs_), partB)

    def apply_and_pap():
        def g_body(g, accs):
            base = g * 16
            pbs = [ps[b, pl.ds(base, 16)] for b in range(B)]
            ss = [zero16] * B
            for k in range(K):
                wv = wbuf[k, pl.ds(base, 16)]
                nlv = nlbuf[k, pl.ds(base, 16)]
                for b in range(B):
                    pn = plsc.load_gather(
                        pfull, [nlv if b == 0 else nlv + b])
                    ss[b] = ss[b] + wv * pn
            dv = degbuf[pl.ds(base, 16)]
            new = []
            for b in range(B):
                apv = pbs[b] + mus * (dv * pbs[b] - ss[b])
                ap[b, pl.ds(base, 16)] = apv
                new.append(accs[b] + pbs[b] * apv)
            return tuple(new)
        accs = lax.fori_loop(0, NG, g_body, (zero16,) * B)
        return [jnp.sum(a) for a in accs]

    lane4 = lane * 4

    def cg_iter(it, carry):
        rs0, rs1, rs2, rs3, par = carry
        rsv = [rs0, rs1, rs2, rs3]
        pap = global_reduce(apply_and_pap(), partA)
        alpha = [rsv[b] / (pap[b] + 1e-12) for b in range(B)]

        def upd_body(g, accs):
            base = g * 16
            out = []
            for b in range(B):
                pv = ps[b, pl.ds(base, 16)]
                av = ap[b, pl.ds(base, 16)]
                xs[b, pl.ds(base, 16)] = xs[b, pl.ds(base, 16)] + alpha[b] * pv
                rv = rs_[b, pl.ds(base, 16)] - alpha[b] * av
                rs_[b, pl.ds(base, 16)] = rv
                out.append(accs[b] + rv * rv)
            return tuple(out)
        rsn_part = lax.fori_loop(0, NG, upd_body, (zero16,) * B)
        rsn = global_reduce([jnp.sum(a) for a in rsn_part], partB)
        beta = [rsn[b] / (rsv[b] + 1e-12) for b in range(B)]
        newpar = 1 - par

        def pupd_body(g, _):
            base = g * 16
            idx0 = (base * 4) + lane4
            for b in range(B):
                pv = rs_[b, pl.ds(base, 16)] + beta[b] * ps[b, pl.ds(base, 16)]
                ps[b, pl.ds(base, 16)] = pv
                plsc.store_scatter(ptbuf, [idx0 if b == 0 else idx0 + b], pv)
            return 0
        lax.fori_loop(0, NG, pupd_body, 0)

        pltpu.sync_copy(ptbuf, p_sp.at[newpar, pl.ds(n0 * B, NT3 * B)])
        plsc.subcore_barrier()
        pltpu.sync_copy(p_sp.at[newpar], pfull)
        return (rsn[0], rsn[1], rsn[2], rsn[3], newpar)

    lax.fori_loop(
        0, CG_ITERS, cg_iter,
        (rs_vecs[0], rs_vecs[1], rs_vecs[2], rs_vecs[3], jnp.int32(0)))

    @pl.when(cid == 0)
    def _():
        pltpu.sync_copy(xs, xout.at[:, pl.ds(n0, NT3)])


def kernel(x, node_embeddings, fc_weight, fc_bias, theta, neighbor_list):
    x = x.astype(jnp.float32)
    emb_pad = jnp.zeros((NPAD, EMB), jnp.float32).at[:N].set(node_embeddings)
    x_pad8 = jnp.zeros((8, NPAD), jnp.float32).at[:B, :N].set(x)
    w1t = jnp.asarray(fc_weight[:, 1:].T, jnp.float32)
    w0 = fc_weight[:, 0].reshape(1, FD).astype(jnp.float32)
    bias = fc_bias.reshape(1, FD).astype(jnp.float32)

    nl_pad = jnp.zeros((NPAD, K), jnp.int32).at[:N].set(
        neighbor_list.astype(jnp.int32))
    nlg = nl_pad.reshape(T2, NCHUNK, CH * K)
    nl4 = (nl_pad.T * 4).reshape(K, T3, NT3).transpose(1, 0, 2)

    params = jnp.zeros((16,), jnp.float32).at[0].set(
        -1.0 / (2.0 * theta.astype(jnp.float32)))

    ftab = _k1(emb_pad, x_pad8, w1t, w0, bias)
    w2 = _k2(ftab, nlg, params)

    x_pad = x_pad8[:B]
    xT_flat = x_pad.T.reshape(-1)
    xout = _k3(x_pad, xT_flat, w2, nl4)
    return xout[:, :N]


# speedup vs baseline: 1.0169x; 1.0169x over previous
import functools

import jax
import jax.numpy as jnp
from jax import lax
from jax.experimental import pallas as pl
from jax.experimental.pallas import tpu as pltpu
from jax.experimental.pallas import tpu_sc as plsc

N = 10000
K = 16
B = 4
EMB = 128
FD = 64
C = 8.0
MU = 0.5
CG_ITERS = 10

NPAD = 10240
ROW = B * FD
T2 = 32
NT2 = NPAD // T2
T3 = 16
NT3 = NPAD // T3
CH = 8
NCHUNK = NT2 // CH
K1_BLK = 256


def _k1_body(emb_ref, x_ref, w1t_ref, w0_ref, b_ref, out_ref):
    g = jnp.dot(emb_ref[...], w1t_ref[...],
                preferred_element_type=jnp.float32) + b_ref[...]
    cols = []
    for b in range(B):
        fb = g + x_ref[b][:, None] * w0_ref[...]
        cols.append(jnp.where(fb >= 0, fb, 0.2 * fb))
    out_ref[...] = jnp.concatenate(cols, axis=1)


def _k1(emb_pad, x_pad8, w1t, w0, bias):
    return pl.pallas_call(
        _k1_body,
        grid=(NPAD // K1_BLK,),
        in_specs=[
            pl.BlockSpec((K1_BLK, EMB), lambda i: (i, 0)),
            pl.BlockSpec((8, K1_BLK), lambda i: (0, i)),
            pl.BlockSpec((EMB, FD), lambda i: (0, 0)),
            pl.BlockSpec((1, FD), lambda i: (0, 0)),
            pl.BlockSpec((1, FD), lambda i: (0, 0)),
        ],
        out_specs=pl.BlockSpec((K1_BLK, ROW), lambda i: (i, 0)),
        out_shape=jax.ShapeDtypeStruct((NPAD, ROW), jnp.float32),
    )(emb_pad, x_pad8, w1t, w0, bias)


_MESH2 = plsc.VectorSubcoreMesh(core_axis_name="c", subcore_axis_name="s")


@functools.partial(
    pl.kernel,
    out_type=jax.ShapeDtypeStruct((T2, K, NT2), jnp.float32),
    mesh=_MESH2,
    compiler_params=pltpu.CompilerParams(use_tc_tiling_on_sc=False, needs_layout_passes=False),
    scratch_types=[
        pltpu.VMEM((NCHUNK, CH * K), jnp.int32),
        pltpu.VMEM((3, CH * K, ROW), jnp.float32),
        pltpu.VMEM((3, CH, ROW), jnp.float32),
        pltpu.VMEM((K, NT2), jnp.float32),
        pltpu.VMEM((16,), jnp.float32),
        pltpu.VMEM((B * K,), jnp.float32),
        pltpu.SemaphoreType.DMA((3, 2)),
        pltpu.SemaphoreType.DMA((3,)),
    ],
)
def _k2(ftab, nlg, params, w2, nlbuf, nbbuf, cbuf, wtbuf, pvec, d2buf,
        semn, semc):
    wid = lax.axis_index("s") * 2 + lax.axis_index("c")
    n0 = wid * NT2
    pltpu.sync_copy(nlg.at[wid], nlbuf)
    pltpu.sync_copy(params, pvec)
    inv2t = pvec[...][0]
    lane = lax.iota(jnp.int32, 16)
    m15 = lane == 15
    H = CH * K // 2

    def start(chunk, sl):
        for h in range(2):
            pltpu.make_async_copy(
                ftab.at[nlbuf.at[chunk, pl.ds(h * H, H)]],
                nbbuf.at[sl, pl.ds(h * H, H)], semn.at[sl, h]).start()
        pltpu.make_async_copy(
            ftab.at[pl.ds(n0 + chunk * CH, CH)], cbuf.at[sl],
            semc.at[sl]).start()

    def wait(chunk, sl):
        for h in range(2):
            pltpu.make_async_copy(
                ftab.at[nlbuf.at[chunk, pl.ds(h * H, H)]],
                nbbuf.at[sl, pl.ds(h * H, H)], semn.at[sl, h]).wait()
        pltpu.make_async_copy(
            ftab.at[pl.ds(n0 + chunk * CH, CH)], cbuf.at[sl],
            semc.at[sl]).wait()

    start(0, 0)
    start(1, 1)
    start(2, 2)

    def chunk_body(chunk, _):
        sl = lax.rem(chunk, 3)
        wait(chunk, sl)

        def node_body(i, _):
            cv = [cbuf[sl, i, pl.ds(g * 16, 16)] for g in range(ROW // 16)]
            row0 = i * K
            for k in range(K):
                for b in range(B):
                    acc = None
                    for g in range(4):
                        dv = (nbbuf[sl, row0 + k, pl.ds(b * 64 + g * 16, 16)]
                              - cv[b * 4 + g])
                        acc = dv * dv if acc is None else acc + dv * dv
                    cum = plsc.cumsum(acc)
                    plsc.store_scatter(
                        d2buf, [jnp.full((16,), b * K + k, jnp.int32)],
                        cum, mask=m15)
            es = None
            for b in range(B):
                e = jnp.exp(d2buf[pl.ds(b * K, 16)] * inv2t)
                es = e if es is None else es + e
            w_row = es * 0.25
            nloc = chunk * CH + i
            w_row = jnp.where(n0 + nloc < N, w_row, 0.0)
            plsc.store_scatter(
                wtbuf, [lane, jnp.full((16,), nloc, jnp.int32)], w_row)
            return 0

        lax.fori_loop(0, CH, node_body, 0)

        @pl.when(chunk + 3 < NCHUNK)
        def _():
            start(chunk + 3, sl)
        return 0

    lax.fori_loop(0, NCHUNK, chunk_body, 0)
    pltpu.sync_copy(wtbuf, w2.at[wid])


_MESH3 = plsc.VectorSubcoreMesh(core_axis_name="c", subcore_axis_name="s")


@functools.partial(
    pl.kernel,
    out_type=jax.ShapeDtypeStruct((B, NPAD), jnp.float32),
    mesh=_MESH3,
    compiler_params=pltpu.CompilerParams(use_tc_tiling_on_sc=False, needs_layout_passes=False),
    scratch_types=[
        pltpu.VMEM((K, NT3), jnp.float32),
        pltpu.VMEM((K, NT3), jnp.int32),
        pltpu.VMEM((NPAD * B,), jnp.float32),
        pltpu.VMEM((B, NT3), jnp.float32),
        pltpu.VMEM((B, NT3), jnp.float32),
        pltpu.VMEM((B, NT3), jnp.float32),
        pltpu.VMEM((B, NT3), jnp.float32),
        pltpu.VMEM((NT3,), jnp.float32),
        pltpu.VMEM((NT3 * B,), jnp.float32),
        pltpu.VMEM((16,), jnp.float32),
        pltpu.VMEM((T3, 16), jnp.float32),
        pltpu.VMEM_SHARED((2, NPAD * B), jnp.float32),
        pltpu.VMEM_SHARED((T3, 16), jnp.float32),
        pltpu.VMEM_SHARED((T3, 16), jnp.float32),
    ],
)
def _k3(x_hbm, xT_hbm, w2_hbm, nl4_hbm, xout,
        wbuf, nlbuf, pfull, ps, rs_, xs, ap, degbuf, ptbuf, rowbuf, redbuf,
        p_sp, partA, partB):
    cid = lax.axis_index("c")
    t = lax.axis_index("s")
    n0 = t * NT3
    lane = lax.iota(jnp.int32, 16)
    zero16 = jnp.zeros((16,), jnp.float32)
    NG = NT3 // 16

    pltpu.sync_copy(w2_hbm.at[2 * t], wbuf.at[:, pl.ds(0, NT2)])
    pltpu.sync_copy(w2_hbm.at[2 * t + 1], wbuf.at[:, pl.ds(NT2, NT2)])
    pltpu.sync_copy(nl4_hbm.at[t], nlbuf)
    pltpu.sync_copy(xT_hbm, pfull)
    pltpu.sync_copy(x_hbm.at[:, pl.ds(n0, NT3)], ps)
    pltpu.sync_copy(x_hbm.at[:, pl.ds(n0, NT3)], rs_)

    def zero_body(g, _):
        for b in range(B):
            xs[b, pl.ds(g * 16, 16)] = zero16
        return 0
    lax.fori_loop(0, NG, zero_body, 0)

    def deg_body(g, carry):
        sw2, sd2 = carry
        dv = zero16
        for k in range(K):
            wv = wbuf[k, pl.ds(g * 16, 16)]
            dv = dv + wv
            sw2 = sw2 + wv * wv
        degbuf[pl.ds(g * 16, 16)] = dv
        return (sw2, sd2 + dv * dv)
    sw2, sd2 = lax.fori_loop(0, NG, deg_body, (zero16, zero16))

    def global_reduce(vals, part_sp):
        v = zero16
        for b, val in enumerate(vals):
            v = v + jnp.where(lane == b, val, 0.0)
        rowbuf[...] = v
        pltpu.sync_copy(rowbuf, part_sp.at[t])
        plsc.subcore_barrier()
        pltpu.sync_copy(part_sp, redbuf)
        return [
            jnp.full((16,), jnp.sum(plsc.load_gather(
                redbuf, [lane, jnp.full((16,), b, jnp.int32)])))
            for b in range(len(vals))
        ]

    l2a, l2b = global_reduce([jnp.sum(sw2), jnp.sum(sd2)], partA)
    y = l2a + l2b
    ib = plsc.bitcast(y, jnp.int32)
    ib = 0x5F3759DF - lax.shift_right_logical(ib, 1)
    rt = plsc.bitcast(ib, jnp.float32)
    for _ in range(4):
        rt = rt * (1.5 - 0.5 * y * rt * rt)
    mus = (MU * C) * rt

    def dot_slices(b1, b2):
        def rb(g, acs):
            out = []
            for b in range(B):
                out.append(acs[b] + b1[b, pl.ds(g * 16, 16)] *
                           b2[b, pl.ds(g * 16, 16)])
            return tuple(out)
        acs = lax.fori_loop(0, NG, rb, (zero16,) * B)
        return [jnp.sum(a) for a in acs]

    rs_vecs = global_reduce(dot_slices(rs_, rs_), partB)

    def apply_and_pap():
        def g_body(g, accs):
            base = g * 16
            pbs = [ps[b, pl.ds(base, 16)] for b in range(B)]
            ss = [zero16] * B
            for k in range(K):
                wv = wbuf[k, pl.ds(base, 16)]
                nlv = nlbuf[k, pl.ds(base, 16)]
                for b in range(B):
                    pn = plsc.load_gather(
                        pfull, [nlv if b == 0 else nlv + b])
                    ss[b] = ss[b] + wv * pn
            dv = degbuf[pl.ds(base, 16)]
            new = []
            for b in range(B):
                apv = pbs[b] + mus * (dv * pbs[b] - ss[b])
                ap[b, pl.ds(base, 16)] = apv
                new.append(accs[b] + pbs[b] * apv)
            return tuple(new)
        accs = lax.fori_loop(0, NG, g_body, (zero16,) * B)
        return [jnp.sum(a) for a in accs]

    lane4 = lane * 4

    def cg_iter(it, carry):
        rs0, rs1, rs2, rs3, par = carry
        rsv = [rs0, rs1, rs2, rs3]
        pap = global_reduce(apply_and_pap(), partA)
        alpha = [rsv[b] / (pap[b] + 1e-12) for b in range(B)]

        def upd_body(g, accs):
            base = g * 16
            out = []
            for b in range(B):
                pv = ps[b, pl.ds(base, 16)]
                av = ap[b, pl.ds(base, 16)]
                xs[b, pl.ds(base, 16)] = xs[b, pl.ds(base, 16)] + alpha[b] * pv
                rv = rs_[b, pl.ds(base, 16)] - alpha[b] * av
                rs_[b, pl.ds(base, 16)] = rv
                out.append(accs[b] + rv * rv)
            return tuple(out)
        rsn_part = lax.fori_loop(0, NG, upd_body, (zero16,) * B)
        rsn = global_reduce([jnp.sum(a) for a in rsn_part], partB)
        beta = [rsn[b] / (rsv[b] + 1e-12) for b in range(B)]
        newpar = 1 - par

        def pupd_body(g, _):
            base = g * 16
            idx0 = (base * 4) + lane4
            for b in range(B):
                pv = rs_[b, pl.ds(base, 16)] + beta[b] * ps[b, pl.ds(base, 16)]
                ps[b, pl.ds(base, 16)] = pv
                plsc.store_scatter(ptbuf, [idx0 if b == 0 else idx0 + b], pv)
            return 0
        lax.fori_loop(0, NG, pupd_body, 0)

        pltpu.sync_copy(ptbuf, p_sp.at[newpar, pl.ds(n0 * B, NT3 * B)])
        plsc.subcore_barrier()
        pltpu.sync_copy(p_sp.at[newpar], pfull)
        return (rsn[0], rsn[1], rsn[2], rsn[3], newpar)

    lax.fori_loop(
        0, CG_ITERS, cg_iter,
        (rs_vecs[0], rs_vecs[1], rs_vecs[2], rs_vecs[3], jnp.int32(0)))

    @pl.when(cid == 0)
    def _():
        pltpu.sync_copy(xs, xout.at[:, pl.ds(n0, NT3)])


def kernel(x, node_embeddings, fc_weight, fc_bias, theta, neighbor_list):
    x = x.astype(jnp.float32)
    emb_pad = jnp.zeros((NPAD, EMB), jnp.float32).at[:N].set(node_embeddings)
    x_pad8 = jnp.zeros((8, NPAD), jnp.float32).at[:B, :N].set(x)
    w1t = jnp.asarray(fc_weight[:, 1:].T, jnp.float32)
    w0 = fc_weight[:, 0].reshape(1, FD).astype(jnp.float32)
    bias = fc_bias.reshape(1, FD).astype(jnp.float32)

    nl_pad = jnp.zeros((NPAD, K), jnp.int32).at[:N].set(
        neighbor_list.astype(jnp.int32))
    nlg = nl_pad.reshape(T2, NCHUNK, CH * K)
    nl4 = (nl_pad.T * 4).reshape(K, T3, NT3).transpose(1, 0, 2)

    params = jnp.zeros((16,), jnp.float32).at[0].set(
        -1.0 / (2.0 * theta.astype(jnp.float32)))

    ftab = _k1(emb_pad, x_pad8, w1t, w0, bias)
    w2 = _k2(ftab, nlg, params)

    x_pad = x_pad8[:B]
    xT_flat = x_pad.T.reshape(-1)
    xout = _k3(x_pad, xT_flat, w2, nl4)
    return xout[:, :N]
